# Initial kernel scaffold; baseline (speedup 1.0000x reference)
import jax
import jax.numpy as jnp
from jax.experimental import pallas as pl


def kernel(bag, offsets, v, u_weight, v_weight):
    raise NotImplementedError("write your pallas kernel here")


# speedup vs baseline: 11.0949x; 11.0949x over previous
import functools

import jax
import jax.numpy as jnp
from jax import lax
from jax.experimental import pallas as pl
from jax.experimental.pallas import tpu as pltpu, tpu_sc as plsc

DIM = 64
K = 6
CHUNK = 128
BAGCHUNK = 16


def _make_kernel(total_bag, bs):
    nc, ns = 2, 16
    nw = nc * ns
    bpw = bs // nw
    nsearch = max(1, (bpw + 1).bit_length())
    mesh = plsc.VectorSubcoreMesh(
        core_axis_name="c", subcore_axis_name="s",
        num_cores=nc, num_subcores=ns)

    @functools.partial(
        pl.kernel,
        out_type=jax.ShapeDtypeStruct((bs * K,), jnp.float32),
        mesh=mesh,
        compiler_params=pltpu.CompilerParams(
            needs_layout_passes=False, use_tc_tiling_on_sc=False),
        scratch_types=[
            pltpu.VMEM((bpw + 16,), jnp.int32),
            pltpu.VMEM((bpw,), jnp.int32),
            pltpu.VMEM((CHUNK + 8,), jnp.int32),
            pltpu.VMEM((CHUNK,), jnp.int32),
            pltpu.VMEM((CHUNK, DIM), jnp.float32),
            pltpu.VMEM((bpw, DIM), jnp.float32),
            pltpu.VMEM((bpw,), jnp.float32),
            pltpu.VMEM((bpw * K,), jnp.int32),
            pltpu.VMEM((BAGCHUNK * K, DIM), jnp.float32),
            pltpu.VMEM((bpw * K,), jnp.float32),
            pltpu.SemaphoreType.DMA,
        ],
    )
    def kern(bag_hbm, starts_hbm, ends_hbm, vflat_hbm, u_hbm, vw_hbm, out_hbm,
             starts_v, ends_v, bagraw, idxbuf, rows, accum,
             invbuf, vidx_v, vrows, outv, sem):
        w = lax.axis_index("s") * nc + lax.axis_index("c")
        b0 = pl.multiple_of(w * bpw, 8)
        pltpu.sync_copy(starts_hbm.at[pl.ds(b0, bpw)], starts_v.at[pl.ds(0, bpw)])
        pltpu.sync_copy(ends_hbm.at[pl.ds(b0, bpw)], ends_v)
        starts_v[pl.ds(bpw, 16)] = jnp.full((16,), jnp.int32(2147483647))
        s_beg = starts_v[pl.ds(0, 16)][0]
        s_end = ends_v[pl.ds(bpw - 16, 16)][15]
        d0 = lax.rem(s_beg, 8)
        a0 = s_beg - d0

        zf16 = jnp.zeros((16,), jnp.float32)
        iota16 = lax.iota(jnp.int32, 16)

        def zbody(i, _):
            for kk in range(DIM // 16):
                accum[i, pl.ds(16 * kk, 16)] = zf16
            return 0

        lax.fori_loop(0, bpw, zbody, 0)

        nchunks = lax.div(s_end - s_beg + (CHUNK - 1), CHUNK)

        def chunk_body(c, _):
            p0 = s_beg + CHUNK * c
            off = pl.multiple_of(a0 + CHUNK * c, 8)
            pltpu.sync_copy(bag_hbm.at[pl.ds(off, CHUNK + 8)], bagraw)
            for g in range(CHUNK // 16):
                vals = plsc.load_gather(bagraw, [d0 + 16 * g + iota16])
                idxbuf[pl.ds(16 * g, 16)] = vals
            cp = pltpu.async_copy(u_hbm.at[idxbuf], rows, sem)

            nv = s_end - p0
            segs = []
            for g in range(CHUNK // 16):
                e16 = p0 + 16 * g + iota16
                lo = jnp.zeros((16,), jnp.int32)
                hi = jnp.full((16,), jnp.int32(bpw))
                for _it in range(nsearch):
                    mid = lax.shift_right_arithmetic(lo + hi, 1)
                    sm = plsc.load_gather(starts_v, [mid])
                    pred = sm <= e16
                    lo = jnp.where(pred, mid + 1, lo)
                    hi = jnp.where(pred, hi, mid)
                segs.append(lo - 1)

            cp.wait()
            for g in range(CHUNK // 16):
                seg16 = segs[g]
                for j in range(16):
                    e_loc = 16 * g + j
                    valid = jnp.broadcast_to(e_loc < nv, (16,))
                    sgs = jnp.broadcast_to(seg16[j], (16,))
                    for kk in range(DIM // 16):
                        vals = rows[e_loc, pl.ds(16 * kk, 16)]
                        plsc.addupdate_scatter(
                            accum, [sgs, 16 * kk + iota16], vals, mask=valid)
            return 0

        lax.fori_loop(0, nchunks, chunk_body, 0)

        for g in range(bpw // 16):
            cnt = ends_v[pl.ds(16 * g, 16)] - starts_v[pl.ds(16 * g, 16)]
            cf = cnt.astype(jnp.float32)
            invbuf[pl.ds(16 * g, 16)] = 1.0 / jnp.maximum(cf, 1.0)

        v0 = pl.multiple_of(b0 * K, 8)
        pltpu.sync_copy(vflat_hbm.at[pl.ds(v0, bpw * K)], vidx_v)

        def vchunk(j, _):
            voff = pl.multiple_of(j * (BAGCHUNK * K), 8)
            cp = pltpu.async_copy(
                vw_hbm.at[vidx_v.at[pl.ds(voff, BAGCHUNK * K)]], vrows, sem)
            cp.wait()
            boff = pl.multiple_of(j * BAGCHUNK, 8)
            b16 = boff + iota16
            invv = invbuf[pl.ds(boff, 16)]
            acc = [jnp.zeros((16,), jnp.float32) for _ in range(K)]
            jrows = [K * iota16 + jj for jj in range(K)]

            def dbody(d, acc):
                dsplat = jnp.broadcast_to(d, (16,))
                embd = plsc.load_gather(accum, [b16, dsplat])
                return tuple(
                    acc[jj] + embd * plsc.load_gather(vrows, [jrows[jj], dsplat])
                    for jj in range(K))

            acc = lax.fori_loop(0, DIM, dbody, tuple(acc))
            for jj in range(K):
                pos = b16 * K + jj
                plsc.store_scatter(outv, [pos], acc[jj] * invv)
            return 0

        lax.fori_loop(0, bpw // BAGCHUNK, vchunk, 0)
        pltpu.sync_copy(outv, out_hbm.at[pl.ds(pl.multiple_of(b0 * K, 8), bpw * K)])

    return kern


def kernel(bag, offsets, v, u_weight, v_weight):
    total_bag = bag.shape[0]
    bs = offsets.shape[0]
    bag = bag.astype(jnp.int32)
    offsets = offsets.astype(jnp.int32)
    ends = jnp.concatenate(
        [offsets[1:], jnp.full((1,), total_bag, dtype=jnp.int32)])
    bag_pad = jnp.concatenate([bag, jnp.zeros((256,), jnp.int32)])
    v_flat = v.astype(jnp.int32).reshape(-1)
    kern = _make_kernel(total_bag, bs)
    out = kern(bag_pad, offsets, ends, v_flat, u_weight, v_weight)
    return out.reshape(bs, K)


# speedup vs baseline: 11.6692x; 1.0518x over previous
import functools

import jax
import jax.numpy as jnp
from jax import lax
from jax.experimental import pallas as pl
from jax.experimental.pallas import tpu as pltpu, tpu_sc as plsc

DIM = 64
K = 6
CHUNK = 128
BAGCHUNK = 16


def _make_kernel(total_bag, bs):
    nc, ns = 2, 16
    nw = nc * ns
    bpw = bs // nw
    nsearch = max(1, (bpw + 1).bit_length())
    nvchunks = bpw // BAGCHUNK
    mesh = plsc.VectorSubcoreMesh(
        core_axis_name="c", subcore_axis_name="s",
        num_cores=nc, num_subcores=ns)

    @functools.partial(
        pl.kernel,
        out_type=jax.ShapeDtypeStruct((bs * K,), jnp.float32),
        mesh=mesh,
        compiler_params=pltpu.CompilerParams(
            needs_layout_passes=False, use_tc_tiling_on_sc=False),
        scratch_types=[
            pltpu.VMEM((bpw + 16,), jnp.int32),
            pltpu.VMEM((bpw,), jnp.int32),
            pltpu.VMEM((CHUNK + 8,), jnp.int32),
            pltpu.VMEM((CHUNK + 8,), jnp.int32),
            pltpu.VMEM((CHUNK,), jnp.int32),
            pltpu.VMEM((CHUNK,), jnp.int32),
            pltpu.VMEM((CHUNK, DIM), jnp.float32),
            pltpu.VMEM((CHUNK, DIM), jnp.float32),
            pltpu.VMEM((bpw, DIM), jnp.float32),
            pltpu.VMEM((bpw,), jnp.float32),
            pltpu.VMEM((bpw * K,), jnp.int32),
            pltpu.VMEM((BAGCHUNK * K, DIM), jnp.float32),
            pltpu.VMEM((BAGCHUNK * K, DIM), jnp.float32),
            pltpu.VMEM((bpw * K,), jnp.float32),
            pltpu.SemaphoreType.DMA,
            pltpu.SemaphoreType.DMA,
            pltpu.SemaphoreType.DMA,
            pltpu.SemaphoreType.DMA,
        ],
    )
    def kern(bag_hbm, starts_hbm, ends_hbm, vflat_hbm, u_hbm, vw_hbm, out_hbm,
             starts_v, ends_v, bagraw_a, bagraw_b, idx_a, idx_b,
             rows_a, rows_b, accum, invbuf, vidx_v, vrows_a, vrows_b, outv,
             sem_a, sem_b, sem_va, sem_vb):
        w = lax.axis_index("s") * nc + lax.axis_index("c")
        b0 = pl.multiple_of(w * bpw, 8)
        pltpu.sync_copy(starts_hbm.at[pl.ds(b0, bpw)],
                        starts_v.at[pl.ds(0, bpw)])
        pltpu.sync_copy(ends_hbm.at[pl.ds(b0, bpw)], ends_v)
        starts_v[pl.ds(bpw, 16)] = jnp.full((16,), jnp.int32(2147483647))
        s_beg = starts_v[pl.ds(0, 16)][0]
        s_end = ends_v[pl.ds(bpw - 16, 16)][15]
        d0 = lax.rem(s_beg, 8)
        a0 = s_beg - d0
        nchunks = lax.div(s_end - s_beg + (CHUNK - 1), CHUNK)

        zf16 = jnp.zeros((16,), jnp.float32)
        iota16 = lax.iota(jnp.int32, 16)

        v0 = pl.multiple_of(b0 * K, 8)
        pltpu.sync_copy(vflat_hbm.at[pl.ds(v0, bpw * K)], vidx_v)
        vcp0 = pltpu.async_copy(
            vw_hbm.at[vidx_v.at[pl.ds(0, BAGCHUNK * K)]], vrows_a, sem_va)

        def stage(c, bagraw, idxbuf, sem, rows):
            off = pl.multiple_of(a0 + CHUNK * c, 8)
            pltpu.sync_copy(bag_hbm.at[pl.ds(off, CHUNK + 8)], bagraw)
            for g in range(CHUNK // 16):
                vals = plsc.load_gather(bagraw, [d0 + 16 * g + iota16])
                idxbuf[pl.ds(16 * g, 16)] = vals
            return pltpu.async_copy(u_hbm.at[idxbuf], rows, sem)

        @pl.when(nchunks > 0)
        def _():
            stage(0, bagraw_a, idx_a, sem_a, rows_a)

        def zbody(i, _):
            for kk in range(DIM // 16):
                accum[i, pl.ds(16 * kk, 16)] = zf16
            return 0

        lax.fori_loop(0, bpw, zbody, 0)

        def process(c, rows):
            p0 = s_beg + CHUNK * c
            nv = s_end - p0

            def gbody(g, _):
                e16 = p0 + 16 * g + iota16
                lo = jnp.zeros((16,), jnp.int32)
                hi = jnp.full((16,), jnp.int32(bpw))
                for _it in range(nsearch):
                    mid = lax.shift_right_arithmetic(lo + hi, 1)
                    sm = plsc.load_gather(starts_v, [mid])
                    pred = sm <= e16
                    lo = jnp.where(pred, mid + 1, lo)
                    hi = jnp.where(pred, hi, mid)
                seg16 = lo - 1
                ebase = 16 * g
                for j in range(16):
                    e_loc = ebase + j
                    valid = jnp.broadcast_to(e_loc < nv, (16,))
                    sgs = jnp.broadcast_to(seg16[j], (16,))
                    esp = jnp.broadcast_to(e_loc, (16,))
                    for kk in range(DIM // 16):
                        vals = plsc.load_gather(rows, [esp, 16 * kk + iota16])
                        plsc.addupdate_scatter(
                            accum, [sgs, 16 * kk + iota16], vals, mask=valid)
                return 0

            lax.fori_loop(0, CHUNK // 16, gbody, 0)

        npairs = lax.div(nchunks + 1, 2)

        def pair_body(t, _):
            c0 = 2 * t
            c1 = c0 + 1
            c2 = c0 + 2

            @pl.when(c1 < nchunks)
            def _():
                stage(c1, bagraw_b, idx_b, sem_b, rows_b)

            pltpu.make_async_copy(u_hbm.at[idx_a], rows_a, sem_a).wait()
            process(c0, rows_a)

            @pl.when(c2 < nchunks)
            def _():
                stage(c2, bagraw_a, idx_a, sem_a, rows_a)

            @pl.when(c1 < nchunks)
            def _():
                pltpu.make_async_copy(u_hbm.at[idx_b], rows_b, sem_b).wait()
                process(c1, rows_b)

            return 0

        lax.fori_loop(0, npairs, pair_body, 0)

        for g in range(bpw // 16):
            cnt = ends_v[pl.ds(16 * g, 16)] - starts_v[pl.ds(16 * g, 16)]
            cf = cnt.astype(jnp.float32)
            invbuf[pl.ds(16 * g, 16)] = 1.0 / jnp.maximum(cf, 1.0)

        def vstage(j, vrows, sem):
            voff = pl.multiple_of(j * (BAGCHUNK * K), 8)
            return pltpu.async_copy(
                vw_hbm.at[vidx_v.at[pl.ds(voff, BAGCHUNK * K)]], vrows, sem)

        def vprocess(j, vrows):
            boff = pl.multiple_of(j * BAGCHUNK, 8)
            b16 = boff + iota16
            invv = invbuf[pl.ds(boff, 16)]
            acc = tuple(jnp.zeros((16,), jnp.float32) for _ in range(K))
            jrows = [K * iota16 + jj for jj in range(K)]

            def dbody(d, acc):
                dsplat = jnp.broadcast_to(d, (16,))
                embd = plsc.load_gather(accum, [b16, dsplat])
                return tuple(
                    acc[jj] + embd * plsc.load_gather(vrows, [jrows[jj], dsplat])
                    for jj in range(K))

            acc = lax.fori_loop(0, DIM, dbody, acc)
            for jj in range(K):
                pos = b16 * K + jj
                plsc.store_scatter(outv, [pos], acc[jj] * invv)

        def vpair(t, _):
            j0 = 2 * t
            j1 = j0 + 1
            j2 = j0 + 2
            vstage(j1, vrows_b, sem_vb)
            pltpu.make_async_copy(
                vw_hbm.at[vidx_v.at[
                    pl.ds(pl.multiple_of(j0 * (BAGCHUNK * K), 8),
                          BAGCHUNK * K)]],
                vrows_a, sem_va).wait()
            vprocess(j0, vrows_a)

            @pl.when(j2 < nvchunks)
            def _():
                vstage(j2, vrows_a, sem_va)

            pltpu.make_async_copy(
                vw_hbm.at[vidx_v.at[
                    pl.ds(pl.multiple_of(j1 * (BAGCHUNK * K), 8),
                          BAGCHUNK * K)]],
                vrows_b, sem_vb).wait()
            vprocess(j1, vrows_b)
            return 0

        lax.fori_loop(0, nvchunks // 2, vpair, 0)
        pltpu.sync_copy(outv, out_hbm.at[pl.ds(v0, bpw * K)])

    return kern


def kernel(bag, offsets, v, u_weight, v_weight):
    total_bag = bag.shape[0]
    bs = offsets.shape[0]
    bag = bag.astype(jnp.int32)
    offsets = offsets.astype(jnp.int32)
    ends = jnp.concatenate(
        [offsets[1:], jnp.full((1,), total_bag, dtype=jnp.int32)])
    bag_pad = jnp.concatenate([bag, jnp.zeros((256,), jnp.int32)])
    v_flat = v.astype(jnp.int32).reshape(-1)
    kern = _make_kernel(total_bag, bs)
    out = kern(bag_pad, offsets, ends, v_flat, u_weight, v_weight)
    return out.reshape(bs, K)
